# Initial kernel scaffold; baseline (speedup 1.0000x reference)
import functools

import jax
import jax.numpy as jnp
from jax import lax
from jax.experimental import pallas as pl
from jax.experimental.pallas import tpu as pltpu
from jax.experimental.pallas import tpu_sc as plsc

N = 10000
E = 320000
NFEAT = 128
NHID = 128
NCLASS = 40

NPAD = 10240
RPT = NPAD // 16
D2 = 64
CHUNK = 128
NW = 32
CPW = -(-E // (NW * CHUNK))
EPW = CPW * CHUNK
EPAD = EPW * NW

_mesh = plsc.VectorSubcoreMesh(core_axis_name="c", subcore_axis_name="s")


def _worker(c, s):
    return s * 2 + c


def _sc_degree(dst3, zeros16, ones16):
    @functools.partial(
        pl.kernel,
        out_type=jax.ShapeDtypeStruct((2 * NPAD, 16), jnp.float32),
        mesh=_mesh,
        scratch_types=[
            pltpu.VMEM((CPW, CHUNK), jnp.int32),
            pltpu.VMEM((CHUNK, 16), jnp.float32),
            pltpu.VMEM_SHARED((NPAD, 16), jnp.float32),
        ],
    )
    def k(dst_hbm, z_hbm, ones_hbm, out_hbm, dst_v, ones_v, acc_s):
        c = lax.axis_index("c")
        s = lax.axis_index("s")
        w = _worker(c, s)
        pltpu.sync_copy(z_hbm.at[pl.ds(s * RPT, RPT)],
                        acc_s.at[pl.ds(s * RPT, RPT)])
        pltpu.sync_copy(dst_hbm.at[w], dst_v)
        pltpu.sync_copy(ones_hbm, ones_v)
        plsc.subcore_barrier()

        def body(j, _):
            pltpu.sync_copy(ones_v, acc_s.at[dst_v.at[j]], add=True)
            return 0

        lax.fori_loop(0, CPW, body, 0)
        plsc.subcore_barrier()
        pltpu.sync_copy(acc_s.at[pl.ds(s * RPT, RPT)],
                        out_hbm.at[pl.ds(c * NPAD + s * RPT, RPT)])

    return k(dst3, zeros16, ones16)


def _sc_aggregate(hs, src3, dst3, zeros, d):
    @functools.partial(
        pl.kernel,
        out_type=jax.ShapeDtypeStruct((2 * NPAD, d), jnp.float32),
        mesh=_mesh,
        scratch_types=[
            pltpu.VMEM((CPW, CHUNK), jnp.int32),
            pltpu.VMEM((CPW, CHUNK), jnp.int32),
            pltpu.VMEM((CHUNK, d), jnp.float32),
            pltpu.VMEM_SHARED((NPAD, d), jnp.float32),
            pltpu.SemaphoreType.DMA,
        ],
    )
    def k(hs_hbm, src_hbm, dst_hbm, z_hbm, out_hbm,
          src_v, dst_v, rows_v, acc_s, sem):
        c = lax.axis_index("c")
        s = lax.axis_index("s")
        w = _worker(c, s)
        pltpu.sync_copy(z_hbm.at[pl.ds(s * RPT, RPT)],
                        acc_s.at[pl.ds(s * RPT, RPT)])
        pltpu.sync_copy(src_hbm.at[w], src_v)
        pltpu.sync_copy(dst_hbm.at[w], dst_v)
        plsc.subcore_barrier()

        def body(j, _):
            pltpu.async_copy(hs_hbm.at[src_v.at[j]], rows_v, sem).wait()
            pltpu.sync_copy(rows_v, acc_s.at[dst_v.at[j]], add=True)
            return 0

        lax.fori_loop(0, CPW, body, 0)
        plsc.subcore_barrier()
        pltpu.sync_copy(acc_s.at[pl.ds(s * RPT, RPT)],
                        out_hbm.at[pl.ds(c * NPAD + s * RPT, RPT)])

    return k(hs, src3, dst3, zeros)


_BLK = 1024
_GRID = NPAD // _BLK


def _tc_scale_matmul(d0, d1, x, W1):
    def body(d0_r, d1_r, x_r, w_r, hs_r, dinv_r):
        deg = d0_r[:, 0:1] + d1_r[:, 0:1] + 1.0
        dinv = lax.rsqrt(deg)
        h = jnp.dot(x_r[...], w_r[...], preferred_element_type=jnp.float32,
                    precision=lax.Precision.HIGHEST)
        hs_r[...] = h * dinv
        dinv_r[...] = jnp.broadcast_to(dinv, (_BLK, 16))

    return pl.pallas_call(
        body,
        grid=(_GRID,),
        in_specs=[
            pl.BlockSpec((_BLK, 16), lambda i: (i, 0)),
            pl.BlockSpec((_BLK, 16), lambda i: (i, 0)),
            pl.BlockSpec((_BLK, NFEAT), lambda i: (i, 0)),
            pl.BlockSpec((NFEAT, NHID), lambda i: (0, 0)),
        ],
        out_specs=[
            pl.BlockSpec((_BLK, NHID), lambda i: (i, 0)),
            pl.BlockSpec((_BLK, 16), lambda i: (i, 0)),
        ],
        out_shape=[
            jax.ShapeDtypeStruct((NPAD, NHID), jnp.float32),
            jax.ShapeDtypeStruct((NPAD, 16), jnp.float32),
        ],
    )(d0, d1, x, W1)


def _tc_layer1_finish(p0, p1, hs1, dinv16, b1, W2p):
    def body(p0_r, p1_r, hs_r, di_r, b1_r, w_r, o_r):
        dinv = di_r[:, 0:1]
        agg = (p0_r[...] + p1_r[...] + hs_r[...]) * dinv + b1_r[...]
        o1 = jnp.maximum(agg, 0.0)
        h2 = jnp.dot(o1, w_r[...], preferred_element_type=jnp.float32,
                     precision=lax.Precision.HIGHEST)
        o_r[...] = h2 * dinv

    return pl.pallas_call(
        body,
        grid=(_GRID,),
        in_specs=[
            pl.BlockSpec((_BLK, NHID), lambda i: (i, 0)),
            pl.BlockSpec((_BLK, NHID), lambda i: (i, 0)),
            pl.BlockSpec((_BLK, NHID), lambda i: (i, 0)),
            pl.BlockSpec((_BLK, 16), lambda i: (i, 0)),
            pl.BlockSpec((1, NHID), lambda i: (0, 0)),
            pl.BlockSpec((NHID, D2), lambda i: (0, 0)),
        ],
        out_specs=pl.BlockSpec((_BLK, D2), lambda i: (i, 0)),
        out_shape=jax.ShapeDtypeStruct((NPAD, D2), jnp.float32),
    )(p0, p1, hs1, dinv16, b1, W2p)


def _tc_layer2_finish(q0, q1, hs2, dinv16, b2p):
    def body(q0_r, q1_r, hs_r, di_r, b2_r, o_r):
        dinv = di_r[:, 0:1]
        logits = (q0_r[...] + q1_r[...] + hs_r[...]) * dinv + b2_r[...]
        mask = lax.broadcasted_iota(jnp.int32, (_BLK, D2), 1) < NCLASS
        masked = jnp.where(mask, logits, -jnp.inf)
        m = jnp.max(masked, axis=1, keepdims=True)
        ex = jnp.where(mask, jnp.exp(logits - m), 0.0)
        lse = jnp.log(jnp.sum(ex, axis=1, keepdims=True))
        o_r[...] = logits - m - lse

    return pl.pallas_call(
        body,
        grid=(_GRID,),
        in_specs=[
            pl.BlockSpec((_BLK, D2), lambda i: (i, 0)),
            pl.BlockSpec((_BLK, D2), lambda i: (i, 0)),
            pl.BlockSpec((_BLK, D2), lambda i: (i, 0)),
            pl.BlockSpec((_BLK, 16), lambda i: (i, 0)),
            pl.BlockSpec((1, D2), lambda i: (0, 0)),
        ],
        out_specs=pl.BlockSpec((_BLK, D2), lambda i: (i, 0)),
        out_shape=jax.ShapeDtypeStruct((NPAD, D2), jnp.float32),
    )(q0, q1, hs2, dinv16, b2p)


def kernel(x, edge_index, W1, b1, W2, b2):
    src = edge_index[0].astype(jnp.int32)
    dst = edge_index[1].astype(jnp.int32)
    pad = jnp.full((EPAD - E,), N, jnp.int32)
    src3 = jnp.concatenate([src, pad]).reshape(NW, CPW, CHUNK)
    dst3 = jnp.concatenate([dst, pad]).reshape(NW, CPW, CHUNK)

    xp = jnp.zeros((NPAD, NFEAT), jnp.float32).at[:N].set(x)
    W2p = jnp.zeros((NHID, D2), jnp.float32).at[:, :NCLASS].set(W2)
    b1r = b1.reshape(1, NHID)
    b2r = jnp.zeros((1, D2), jnp.float32).at[0, :NCLASS].set(b2)

    zeros16 = jnp.zeros((NPAD, 16), jnp.float32)
    zeros128 = jnp.zeros((NPAD, NHID), jnp.float32)
    zeros64 = jnp.zeros((NPAD, D2), jnp.float32)
    ones16 = jnp.ones((CHUNK, 16), jnp.float32)

    degp = _sc_degree(dst3, zeros16, ones16)
    hs1, dinv16 = _tc_scale_matmul(degp[:NPAD], degp[NPAD:], xp, W1)
    p = _sc_aggregate(hs1, src3, dst3, zeros128, NHID)
    hs2 = _tc_layer1_finish(p[:NPAD], p[NPAD:], hs1, dinv16, b1r, W2p)
    q = _sc_aggregate(hs2, src3, dst3, zeros64, D2)
    out = _tc_layer2_finish(q[:NPAD], q[NPAD:], hs2, dinv16, b2r)
    return out[:N, :NCLASS]


# speedup vs baseline: 13.0301x; 13.0301x over previous
import functools

import jax
import jax.numpy as jnp
from jax import lax
from jax.experimental import pallas as pl
from jax.experimental.pallas import tpu as pltpu
from jax.experimental.pallas import tpu_sc as plsc

N = 10000
E = 320000
NFEAT = 128
NHID = 128
NCLASS = 40

NPAD = 10240
RPT = NPAD // 16
D2 = 128
CHUNK = 128
NW = 32
CPW = -(-E // (NW * CHUNK))
EPW = CPW * CHUNK
EPAD = EPW * NW

_mesh = plsc.VectorSubcoreMesh(core_axis_name="c", subcore_axis_name="s")


def _worker(c, s):
    return s * 2 + c


def _sc_degree(dst3, zeros16, ones16):
    @functools.partial(
        pl.kernel,
        out_type=jax.ShapeDtypeStruct((2 * NPAD, 16), jnp.float32),
        mesh=_mesh,
        scratch_types=[
            pltpu.VMEM((CPW, CHUNK), jnp.int32),
            pltpu.VMEM((CHUNK, 16), jnp.float32),
            pltpu.VMEM_SHARED((NPAD, 16), jnp.float32),
        ],
    )
    def k(dst_hbm, z_hbm, ones_hbm, out_hbm, dst_v, ones_v, acc_s):
        c = lax.axis_index("c")
        s = lax.axis_index("s")
        w = _worker(c, s)
        pltpu.sync_copy(z_hbm.at[pl.ds(s * RPT, RPT)],
                        acc_s.at[pl.ds(s * RPT, RPT)])
        pltpu.sync_copy(dst_hbm.at[w], dst_v)
        pltpu.sync_copy(ones_hbm, ones_v)
        plsc.subcore_barrier()

        def body(j, _):
            pltpu.sync_copy(ones_v, acc_s.at[dst_v.at[j]], add=True)
            return 0

        lax.fori_loop(0, CPW, body, 0)
        plsc.subcore_barrier()
        pltpu.sync_copy(acc_s.at[pl.ds(s * RPT, RPT)],
                        out_hbm.at[pl.ds(c * NPAD + s * RPT, RPT)])

    return k(dst3, zeros16, ones16)


def _sc_aggregate(hs, src3, dst3, zeros, d):
    @functools.partial(
        pl.kernel,
        out_type=jax.ShapeDtypeStruct((2 * NPAD, d), jnp.float32),
        mesh=_mesh,
        scratch_types=[
            pltpu.VMEM((CPW, CHUNK), jnp.int32),
            pltpu.VMEM((CPW, CHUNK), jnp.int32),
            pltpu.VMEM((CHUNK, d), jnp.float32),
            pltpu.VMEM_SHARED((NPAD, d), jnp.float32),
            pltpu.SemaphoreType.DMA,
        ],
    )
    def k(hs_hbm, src_hbm, dst_hbm, z_hbm, out_hbm,
          src_v, dst_v, rows_v, acc_s, sem):
        c = lax.axis_index("c")
        s = lax.axis_index("s")
        w = _worker(c, s)
        pltpu.sync_copy(z_hbm.at[pl.ds(s * RPT, RPT)],
                        acc_s.at[pl.ds(s * RPT, RPT)])
        pltpu.sync_copy(src_hbm.at[w], src_v)
        pltpu.sync_copy(dst_hbm.at[w], dst_v)
        plsc.subcore_barrier()

        def body(j, _):
            pltpu.async_copy(hs_hbm.at[src_v.at[j]], rows_v, sem).wait()
            pltpu.sync_copy(rows_v, acc_s.at[dst_v.at[j]], add=True)
            return 0

        lax.fori_loop(0, CPW, body, 0)
        plsc.subcore_barrier()
        pltpu.sync_copy(acc_s.at[pl.ds(s * RPT, RPT)],
                        out_hbm.at[pl.ds(c * NPAD + s * RPT, RPT)])

    return k(hs, src3, dst3, zeros)


_BLK = 1024
_GRID = NPAD // _BLK


def _tc_scale_matmul(d0, d1, x, W1):
    def body(d0_r, d1_r, x_r, w_r, hs_r, dinv_r):
        deg = d0_r[:, 0:1] + d1_r[:, 0:1] + 1.0
        dinv = lax.rsqrt(deg)
        h = jnp.dot(x_r[...], w_r[...], preferred_element_type=jnp.float32,
                    precision=lax.Precision.HIGHEST)
        hs_r[...] = h * dinv
        dinv_r[...] = jnp.broadcast_to(dinv, (_BLK, 16))

    return pl.pallas_call(
        body,
        grid=(_GRID,),
        in_specs=[
            pl.BlockSpec((_BLK, 16), lambda i: (i, 0)),
            pl.BlockSpec((_BLK, 16), lambda i: (i, 0)),
            pl.BlockSpec((_BLK, NFEAT), lambda i: (i, 0)),
            pl.BlockSpec((NFEAT, NHID), lambda i: (0, 0)),
        ],
        out_specs=[
            pl.BlockSpec((_BLK, NHID), lambda i: (i, 0)),
            pl.BlockSpec((_BLK, 16), lambda i: (i, 0)),
        ],
        out_shape=[
            jax.ShapeDtypeStruct((NPAD, NHID), jnp.float32),
            jax.ShapeDtypeStruct((NPAD, 16), jnp.float32),
        ],
    )(d0, d1, x, W1)


def _tc_layer1_finish(p0, p1, hs1, dinv16, b1, W2p):
    def body(p0_r, p1_r, hs_r, di_r, b1_r, w_r, o_r):
        dinv = di_r[:, 0:1]
        agg = (p0_r[...] + p1_r[...] + hs_r[...]) * dinv + b1_r[...]
        o1 = jnp.maximum(agg, 0.0)
        h2 = jnp.dot(o1, w_r[...], preferred_element_type=jnp.float32,
                     precision=lax.Precision.HIGHEST)
        o_r[...] = h2 * dinv

    return pl.pallas_call(
        body,
        grid=(_GRID,),
        in_specs=[
            pl.BlockSpec((_BLK, NHID), lambda i: (i, 0)),
            pl.BlockSpec((_BLK, NHID), lambda i: (i, 0)),
            pl.BlockSpec((_BLK, NHID), lambda i: (i, 0)),
            pl.BlockSpec((_BLK, 16), lambda i: (i, 0)),
            pl.BlockSpec((1, NHID), lambda i: (0, 0)),
            pl.BlockSpec((NHID, D2), lambda i: (0, 0)),
        ],
        out_specs=pl.BlockSpec((_BLK, D2), lambda i: (i, 0)),
        out_shape=jax.ShapeDtypeStruct((NPAD, D2), jnp.float32),
    )(p0, p1, hs1, dinv16, b1, W2p)


def _tc_layer2_finish(q0, q1, hs2, dinv16, b2p):
    def body(q0_r, q1_r, hs_r, di_r, b2_r, o_r):
        dinv = di_r[:, 0:1]
        logits = (q0_r[...] + q1_r[...] + hs_r[...]) * dinv + b2_r[...]
        mask = lax.broadcasted_iota(jnp.int32, (_BLK, D2), 1) < NCLASS
        masked = jnp.where(mask, logits, -jnp.inf)
        m = jnp.max(masked, axis=1, keepdims=True)
        ex = jnp.where(mask, jnp.exp(logits - m), 0.0)
        lse = jnp.log(jnp.sum(ex, axis=1, keepdims=True))
        o_r[...] = logits - m - lse

    return pl.pallas_call(
        body,
        grid=(_GRID,),
        in_specs=[
            pl.BlockSpec((_BLK, D2), lambda i: (i, 0)),
            pl.BlockSpec((_BLK, D2), lambda i: (i, 0)),
            pl.BlockSpec((_BLK, D2), lambda i: (i, 0)),
            pl.BlockSpec((_BLK, 16), lambda i: (i, 0)),
            pl.BlockSpec((1, D2), lambda i: (0, 0)),
        ],
        out_specs=pl.BlockSpec((_BLK, D2), lambda i: (i, 0)),
        out_shape=jax.ShapeDtypeStruct((NPAD, D2), jnp.float32),
    )(q0, q1, hs2, dinv16, b2p)


def kernel(x, edge_index, W1, b1, W2, b2):
    src = edge_index[0].astype(jnp.int32)
    dst = edge_index[1].astype(jnp.int32)
    pad = jnp.full((EPAD - E,), N, jnp.int32)
    src3 = jnp.concatenate([src, pad]).reshape(NW, CPW, CHUNK)
    dst3 = jnp.concatenate([dst, pad]).reshape(NW, CPW, CHUNK)

    xp = jnp.zeros((NPAD, NFEAT), jnp.float32).at[:N].set(x)
    W2p = jnp.zeros((NHID, D2), jnp.float32).at[:, :NCLASS].set(W2)
    b1r = b1.reshape(1, NHID)
    b2r = jnp.zeros((1, D2), jnp.float32).at[0, :NCLASS].set(b2)

    zeros16 = jnp.zeros((NPAD, 16), jnp.float32)
    zeros128 = jnp.zeros((NPAD, NHID), jnp.float32)
    zeros64 = jnp.zeros((NPAD, D2), jnp.float32)
    ones16 = jnp.ones((CHUNK, 16), jnp.float32)

    degp = _sc_degree(dst3, zeros16, ones16)
    hs1, dinv16 = _tc_scale_matmul(degp[:NPAD], degp[NPAD:], xp, W1)
    p = _sc_aggregate(hs1, src3, dst3, zeros128, NHID)
    hs2 = _tc_layer1_finish(p[:NPAD], p[NPAD:], hs1, dinv16, b1r, W2p)
    q = _sc_aggregate(hs2, src3, dst3, zeros64, D2)
    out = _tc_layer2_finish(q[:NPAD], q[NPAD:], hs2, dinv16, b2r)
    return out[:N, :NCLASS]
